# baseline (speedup 1.0000x reference)
import functools

import jax
import jax.numpy as jnp
from jax import lax
from jax.experimental import pallas as pl
from jax.experimental.pallas import tpu as pltpu
from jax.experimental.pallas import tpu_sc as plsc

P = 100000
B = 4096
T = 200
C = 32

NC = 2
NS = 16
NW = NC * NS
BPW = B // NW
TC_ = 8
NCHUNK = T // TC_
TILE_W = C * BPW

_mesh = plsc.VectorSubcoreMesh(core_axis_name="c", subcore_axis_name="s")


@functools.partial(
    pl.kernel,
    mesh=_mesh,
    out_type=[
        jax.ShapeDtypeStruct((T, B * C), jnp.float32),
        jax.ShapeDtypeStruct((TILE_W,), jnp.float32),
    ],
    scratch_types=[
        pltpu.VMEM((BPW,), jnp.int32),
        pltpu.VMEM((16,), jnp.int32),
        pltpu.SMEM((BPW,), jnp.int32),
        [pltpu.VMEM((BPW * TC_, C), jnp.float32) for _ in range(2)],
        [pltpu.VMEM((TILE_W,), jnp.float32) for _ in range(2)],
        [pltpu.SemaphoreType.DMA for _ in range(2)],
        [pltpu.SemaphoreType.DMA for _ in range(2)],
    ],
    compiler_params=pltpu.CompilerParams(
        needs_layout_passes=False, use_tc_tiling_on_sc=False
    ),
)
def _sc_gather(idx_hbm, shift_hbm, data_hbm, out_hbm, dummy_hbm,
               idx_v, shift_v, idx_s, stages, tiles, rsems, wsems):
    wid = lax.axis_index("s") * NC + lax.axis_index("c")
    base_b = wid * BPW

    pltpu.sync_copy(idx_hbm.at[pl.ds(base_b, BPW)], idx_v)
    pltpu.sync_copy(shift_hbm, shift_v)
    shift_vec = shift_v[...]

    for g in range(BPW // 16):
        v = idx_v[pl.ds(g * 16, 16)] + shift_vec
        v = jnp.where(v >= P, v - P, v)
        v = jnp.where(v < 0, v + P, v)
        for k in range(16):
            idx_s[g * 16 + k] = v[k]

    iota = lax.iota(jnp.int32, 16)
    rbase = [((iota + d) % 16) * TC_ for d in range(16)]
    wbase = [iota * BPW + (iota + d) % 16 for d in range(16)]
    colv = [iota, iota + 16]

    def fire_reads(g, sb):
        t0 = jnp.minimum(g, NCHUNK - 1) * TC_

        def one(l, carry):
            start = idx_s[l] + t0
            pltpu.async_copy(
                data_hbm.at[pl.ds(start, TC_), pl.ds(0, C)],
                stages[sb].at[pl.ds(l * TC_, TC_)],
                rsems[sb],
            )
            return carry

        lax.fori_loop(0, BPW, one, 0)

    def wait_reads(sb):
        pltpu.make_async_copy(
            data_hbm.at[pl.ds(0, BPW * TC_), pl.ds(0, C)],
            stages[sb], rsems[sb]
        ).wait()

    def wait_tile(tb):
        pltpu.make_async_copy(dummy_hbm, tiles[tb], wsems[tb]).wait()

    def fire_writes(t, tb):
        for ct in range(4):
            pltpu.async_copy(
                tiles[tb].at[pl.ds(ct * 8 * BPW, 8 * BPW)],
                out_hbm.at[t, pl.ds((ct * NW + wid) * 8 * BPW, 8 * BPW)],
                wsems[tb],
            )

    def transpose_t(stage, tile, trel):
        def per_lb(lb, carry):
            for cb in range(2):
                rs = lb * 16 * TC_ + trel
                ws = cb * 16 * BPW + lb * 16
                for h in range(2):
                    vals = [
                        plsc.load_gather(
                            stage, [rbase[h * 8 + k] + rs, colv[cb]]
                        )
                        for k in range(8)
                    ]
                    for k in range(8):
                        plsc.store_scatter(
                            tile, [wbase[h * 8 + k] + ws], vals[k]
                        )
            return carry

        lax.fori_loop(0, BPW // 16, per_lb, 0)

    pltpu.async_copy(tiles[0], dummy_hbm, wsems[0])
    pltpu.async_copy(tiles[1], dummy_hbm, wsems[1])

    fire_reads(0, 0)
    fire_reads(1, 1)

    def chunk_pair(gp, carry):
        for sb in range(2):
            g = 2 * gp + sb
            wait_reads(sb)

            def tpair(j, carry2):
                for tb in range(2):
                    trel = 2 * j + tb
                    wait_tile(tb)
                    transpose_t(stages[sb], tiles[tb], trel)
                    fire_writes(g * TC_ + trel, tb)
                return carry2

            lax.fori_loop(0, TC_ // 2, tpair, 0)
            fire_reads(g + 2, sb)
        return carry

    lax.fori_loop(0, NCHUNK // 2, chunk_pair, 0)

    wait_reads(0)

    def tpair_last(j, carry2):
        for tb in range(2):
            trel = 2 * j + tb
            wait_tile(tb)
            transpose_t(stages[0], tiles[tb], trel)
            fire_writes((NCHUNK - 1) * TC_ + trel, tb)
        return carry2

    lax.fori_loop(0, TC_ // 2, tpair_last, 0)

    wait_reads(1)
    wait_tile(0)
    wait_tile(1)


def kernel(index, length, data):
    shift = jnp.broadcast_to(
        (jnp.asarray(length, jnp.int32) - T).reshape(()), (16,)
    ).astype(jnp.int32)
    data_ext = jnp.concatenate([data, data[:T]], axis=0)
    data_wide = jnp.concatenate(
        [data_ext, data_ext, data_ext, data_ext], axis=1
    )
    out, _ = _sc_gather(index.astype(jnp.int32), shift, data_wide)
    out5 = out.reshape(T, 4, NW, 8, BPW)
    return jnp.transpose(out5, (2, 4, 0, 1, 3)).reshape(B, T, C)


# speedup vs baseline: 1.4889x; 1.4889x over previous
import functools

import jax
import jax.numpy as jnp
from jax import lax
from jax.experimental import pallas as pl
from jax.experimental.pallas import tpu as pltpu
from jax.experimental.pallas import tpu_sc as plsc

P = 100000
B = 4096
T = 200
C = 32

NC = 2
NS = 16
NW = NC * NS
BPW = B // NW
TC_ = 8
NCHUNK = T // TC_
TILE_W = C * BPW

_mesh = plsc.VectorSubcoreMesh(core_axis_name="c", subcore_axis_name="s")


@functools.partial(
    pl.kernel,
    mesh=_mesh,
    out_type=[
        jax.ShapeDtypeStruct((T, B * C), jnp.float32),
        jax.ShapeDtypeStruct((TILE_W,), jnp.float32),
    ],
    scratch_types=[
        pltpu.VMEM((BPW,), jnp.int32),
        pltpu.VMEM((16,), jnp.int32),
        pltpu.SMEM((BPW,), jnp.int32),
        [pltpu.VMEM((BPW * TC_ * C,), jnp.float32) for _ in range(2)],
        [pltpu.VMEM((TILE_W,), jnp.float32) for _ in range(2)],
        [pltpu.SemaphoreType.DMA for _ in range(2)],
        [pltpu.SemaphoreType.DMA for _ in range(2)],
    ],
    compiler_params=pltpu.CompilerParams(
        needs_layout_passes=False, use_tc_tiling_on_sc=False
    ),
)
def _sc_gather(idx_hbm, shift_hbm, data_hbm, out_hbm, dummy_hbm,
               idx_v, shift_v, idx_s, stages, tiles, rsems, wsems):
    wid = lax.axis_index("s") * NC + lax.axis_index("c")
    base_b = wid * BPW

    pltpu.sync_copy(idx_hbm.at[pl.ds(base_b, BPW)], idx_v)
    pltpu.sync_copy(shift_hbm, shift_v)
    shift_vec = shift_v[...]

    for g in range(BPW // 16):
        v = idx_v[pl.ds(g * 16, 16)] + shift_vec
        v = jnp.where(v >= P, v - P, v)
        v = jnp.where(v < 0, v + P, v)
        for k in range(16):
            idx_s[g * 16 + k] = v[k]

    iota = lax.iota(jnp.int32, 16)
    rbase = [((iota + d) % 16) * (TC_ * C) + iota for d in range(16)]
    wbase = [iota * BPW + (iota + d) % 16 for d in range(16)]

    def fire_reads(g, sb):
        t0 = jnp.minimum(g, NCHUNK - 1) * TC_

        def one(l, carry):
            start = (idx_s[l] + t0) * C
            pltpu.async_copy(
                data_hbm.at[pl.ds(start, TC_ * C)],
                stages[sb].at[pl.ds(l * TC_ * C, TC_ * C)],
                rsems[sb],
            )
            return carry

        lax.fori_loop(0, BPW, one, 0)

    def wait_reads(sb):
        pltpu.make_async_copy(
            data_hbm.at[pl.ds(0, BPW * TC_ * C)], stages[sb], rsems[sb]
        ).wait()

    def wait_tile(tb):
        pltpu.make_async_copy(dummy_hbm, tiles[tb], wsems[tb]).wait()

    def fire_writes(t, tb):
        for ct in range(4):
            pltpu.async_copy(
                tiles[tb].at[pl.ds(ct * 8 * BPW, 8 * BPW)],
                out_hbm.at[t, pl.ds((ct * NW + wid) * 8 * BPW, 8 * BPW)],
                wsems[tb],
            )

    def transpose_t(stage, tile, trel):
        @functools.partial(plsc.parallel_loop, 0, BPW // 16)
        def per_lb(lb):
            for cb in range(2):
                rs = lb * 16 * TC_ * C + trel * C + cb * 16
                ws = cb * 16 * BPW + lb * 16
                for h in range(2):
                    vals = [
                        plsc.load_gather(stage, [rbase[h * 8 + k] + rs])
                        for k in range(8)
                    ]
                    for k in range(8):
                        plsc.store_scatter(
                            tile, [wbase[h * 8 + k] + ws], vals[k]
                        )

    pltpu.async_copy(tiles[0], dummy_hbm, wsems[0])
    pltpu.async_copy(tiles[1], dummy_hbm, wsems[1])

    fire_reads(0, 0)
    fire_reads(1, 1)

    def chunk_pair(gp, carry):
        for sb in range(2):
            g = 2 * gp + sb
            wait_reads(sb)

            def tpair(j, carry2):
                for tb in range(2):
                    trel = 2 * j + tb
                    wait_tile(tb)
                    transpose_t(stages[sb], tiles[tb], trel)
                    fire_writes(g * TC_ + trel, tb)
                return carry2

            lax.fori_loop(0, TC_ // 2, tpair, 0)
            fire_reads(g + 2, sb)
        return carry

    lax.fori_loop(0, NCHUNK // 2, chunk_pair, 0)

    wait_reads(0)

    def tpair_last(j, carry2):
        for tb in range(2):
            trel = 2 * j + tb
            wait_tile(tb)
            transpose_t(stages[0], tiles[tb], trel)
            fire_writes((NCHUNK - 1) * TC_ + trel, tb)
        return carry2

    lax.fori_loop(0, TC_ // 2, tpair_last, 0)

    wait_reads(1)
    wait_tile(0)
    wait_tile(1)


def kernel(index, length, data):
    shift = jnp.broadcast_to(
        (jnp.asarray(length, jnp.int32) - T).reshape(()), (16,)
    ).astype(jnp.int32)
    data_flat = data.reshape(-1)
    data_ext = jnp.concatenate([data_flat, data_flat[: T * C]])
    out, _ = _sc_gather(index.astype(jnp.int32), shift, data_ext)
    out5 = out.reshape(T, 4, NW, 8, BPW)
    return jnp.transpose(out5, (2, 4, 0, 1, 3)).reshape(B, T, C)
